# Initial kernel scaffold; baseline (speedup 1.0000x reference)
import jax
import jax.numpy as jnp
from jax.experimental import pallas as pl


def kernel(x):
    raise NotImplementedError("write your pallas kernel here")


# speedup vs baseline: 1.1943x; 1.1943x over previous
import jax
import jax.numpy as jnp
import numpy as np
from jax.experimental import pallas as pl
from jax.experimental.pallas import tpu as pltpu

_A = 32768
_NHEADS = 4
_ROWS = 128
_BLK = 2048
_NB = _A // _BLK

_TINY = np.float32(np.finfo(np.float32).tiny)
_BIG_I32 = np.int32(2**31 - 1)


def _threefry_bits(k1, k2, x1):
  ks2 = k1 ^ k2 ^ np.uint32(0x1BD11BDA)
  ks = (k1, k2, ks2)
  rotations = ((13, 15, 26, 6), (17, 29, 16, 24))

  def rotl(v, d):
    return (v << np.uint32(d)) | (v >> np.uint32(32 - d))

  x0 = jnp.zeros_like(x1) + k1
  x1 = x1 + k2
  for i in range(5):
    for r in rotations[i % 2]:
      x0 = x0 + x1
      x1 = rotl(x1, r)
      x1 = x1 ^ x0
    x0 = x0 + ks[(i + 1) % 3]
    x1 = x1 + ks[(i + 2) % 3] + np.uint32(i + 1)
  return x0 ^ x1


def _sample_kernel(keys_ref, x_ref, out_ref, best_val, best_idx):
  h = pl.program_id(0)
  cb = pl.program_id(1)

  k1 = keys_ref[h, 0]
  k2 = keys_ref[h, 1]

  row = jax.lax.broadcasted_iota(jnp.uint32, (_ROWS, _BLK), 0)
  col = jax.lax.broadcasted_iota(jnp.uint32, (_ROWS, _BLK), 1)
  col_base = (cb * _BLK).astype(jnp.uint32)
  p = row * np.uint32(_A) + col + col_base

  bits = _threefry_bits(k1, k2, p)

  float_bits = (bits >> np.uint32(9)) | np.uint32(0x3F800000)
  f = pltpu.bitcast(float_bits, jnp.float32) - np.float32(1.0)
  u = jnp.maximum(_TINY, f + _TINY)
  g = -jnp.log(-jnp.log(u))

  v = g + x_ref[...]

  m = jnp.max(v, axis=1, keepdims=True)
  colg = jax.lax.broadcasted_iota(jnp.int32, (_ROWS, _BLK), 1) + cb * _BLK
  cand = jnp.where(v == m, colg, _BIG_I32)
  idx = jnp.min(cand, axis=1, keepdims=True)

  @pl.when(cb == 0)
  def _():
    best_val[...] = m
    best_idx[...] = idx

  @pl.when(cb != 0)
  def _():
    better = m > best_val[...]
    best_val[...] = jnp.where(better, m, best_val[...])
    best_idx[...] = jnp.where(better, idx, best_idx[...])

  @pl.when(cb == _NB - 1)
  def _():
    lane = jax.lax.broadcasted_iota(jnp.int32, (_ROWS, _NHEADS), 1)
    out_ref[...] = jnp.where(lane == h, best_idx[...], out_ref[...])


@jax.jit
def kernel(x):
  base = jax.random.key(42)
  keys = jnp.stack(
      [jax.random.key_data(jax.random.fold_in(base, i)) for i in range(_NHEADS)]
  ).astype(jnp.uint32)

  out = pl.pallas_call(
      _sample_kernel,
      grid=(_NHEADS, _NB),
      in_specs=[
          pl.BlockSpec(memory_space=pltpu.SMEM),
          pl.BlockSpec((_ROWS, _BLK), lambda h, cb: (0, h * _NB + cb)),
      ],
      out_specs=pl.BlockSpec((_ROWS, _NHEADS), lambda h, cb: (0, 0)),
      out_shape=jax.ShapeDtypeStruct((_ROWS, _NHEADS), jnp.int32),
      scratch_shapes=[
          pltpu.VMEM((_ROWS, 1), jnp.float32),
          pltpu.VMEM((_ROWS, 1), jnp.int32),
      ],
      compiler_params=pltpu.CompilerParams(
          dimension_semantics=("arbitrary", "arbitrary"),
      ),
  )(keys, x)

  return out.T.reshape(-1)


# speedup vs baseline: 1.2247x; 1.0255x over previous
import jax
import jax.numpy as jnp
import numpy as np
from jax.experimental import pallas as pl
from jax.experimental.pallas import tpu as pltpu

_A = 32768
_NHEADS = 4
_ROWS = 128
_BLK = 2048
_NB = _A // _BLK

_TINY = np.float32(np.finfo(np.float32).tiny)
_BIG_I32 = np.int32(2**31 - 1)


def _threefry_bits(k1, k2, x1):
  ks2 = k1 ^ k2 ^ np.uint32(0x1BD11BDA)
  ks = (k1, k2, ks2)
  rotations = ((13, 15, 26, 6), (17, 29, 16, 24))

  def rotl(v, d):
    return (v << np.uint32(d)) | (v >> np.uint32(32 - d))

  x0 = jnp.zeros_like(x1) + k1
  for i in range(5):
    for r in rotations[i % 2]:
      x0 = x0 + x1
      x1 = rotl(x1, r)
      x1 = x1 ^ x0
    x0 = x0 + ks[(i + 1) % 3]
    x1 = x1 + ks[(i + 2) % 3] + np.uint32(i + 1)
  return x0 ^ x1


def _sample_kernel(keys_ref, x_ref, out_ref, best_val, best_idx, p_base):
  h = pl.program_id(0)
  cb = pl.program_id(1)

  k1 = keys_ref[h, 0]
  k2 = keys_ref[h, 1]

  @pl.when((h == 0) & (cb == 0))
  def _():
    row = jax.lax.broadcasted_iota(jnp.uint32, (_ROWS, _BLK), 0)
    col = jax.lax.broadcasted_iota(jnp.uint32, (_ROWS, _BLK), 1)
    p_base[...] = row * np.uint32(_A) + col

  x1 = p_base[...] + (k2 + (cb * _BLK).astype(jnp.uint32))
  bits = _threefry_bits(k1, k2, x1)

  float_bits = (bits >> np.uint32(9)) | np.uint32(0x3F800000)
  f = pltpu.bitcast(float_bits, jnp.float32) - np.float32(1.0)
  u = jnp.maximum(f, _TINY)
  l2 = jnp.log(-jnp.log(u))
  v = x_ref[...] - l2

  m = jnp.max(v, axis=1, keepdims=True)
  colg = jax.lax.broadcasted_iota(jnp.int32, (_ROWS, _BLK), 1) + cb * _BLK
  cand = jnp.where(v == m, colg, _BIG_I32)
  idx = jnp.min(cand, axis=1, keepdims=True)

  @pl.when(cb == 0)
  def _():
    best_val[...] = m
    best_idx[...] = idx

  @pl.when(cb != 0)
  def _():
    better = m > best_val[...]
    best_val[...] = jnp.where(better, m, best_val[...])
    best_idx[...] = jnp.where(better, idx, best_idx[...])

  @pl.when(cb == _NB - 1)
  def _():
    lane = jax.lax.broadcasted_iota(jnp.int32, (_ROWS, _NHEADS), 1)
    out_ref[...] = jnp.where(lane == h, best_idx[...], out_ref[...])


@jax.jit
def kernel(x):
  base = jax.random.key(42)
  keys = jnp.stack(
      [jax.random.key_data(jax.random.fold_in(base, i)) for i in range(_NHEADS)]
  ).astype(jnp.uint32)

  out = pl.pallas_call(
      _sample_kernel,
      grid=(_NHEADS, _NB),
      in_specs=[
          pl.BlockSpec(memory_space=pltpu.SMEM),
          pl.BlockSpec((_ROWS, _BLK), lambda h, cb: (0, h * _NB + cb)),
      ],
      out_specs=pl.BlockSpec((_ROWS, _NHEADS), lambda h, cb: (0, 0)),
      out_shape=jax.ShapeDtypeStruct((_ROWS, _NHEADS), jnp.int32),
      scratch_shapes=[
          pltpu.VMEM((_ROWS, 1), jnp.float32),
          pltpu.VMEM((_ROWS, 1), jnp.int32),
          pltpu.VMEM((_ROWS, _BLK), jnp.uint32),
      ],
      compiler_params=pltpu.CompilerParams(
          dimension_semantics=("arbitrary", "arbitrary"),
      ),
  )(keys, x)

  return out.T.reshape(-1)
